# Initial kernel scaffold; baseline (speedup 1.0000x reference)
import jax
import jax.numpy as jnp
from jax.experimental import pallas as pl


def kernel(x, type_weight, Wg, A, Bw):
    raise NotImplementedError("write your pallas kernel here")


# speedup vs baseline: 7.2167x; 7.2167x over previous
import jax
import jax.numpy as jnp
from jax.experimental import pallas as pl

_B, _S, _D = 4, 4096, 2048
_E, _K, _R = 8, 2, 8
_SCALE = 16.0 / 8.0
_BLK = 256


def _moe_block(x_ref, tw_ref, wg_ref, aall_ref, ball_ref, out_ref):
    xb = x_ref[...]
    logits = jnp.dot(xb, wg_ref[...],
                     preferred_element_type=jnp.float32)

    e_iota = jax.lax.broadcasted_iota(jnp.int32, (_BLK, _E), 1)
    m1 = jnp.max(logits, axis=1, keepdims=True)
    i1 = jnp.min(jnp.where(logits == m1, e_iota, _E), axis=1, keepdims=True)
    oh1 = e_iota == i1
    masked = jnp.where(oh1, -jnp.inf, logits)
    m2 = jnp.max(masked, axis=1, keepdims=True)
    i2 = jnp.min(jnp.where(masked == m2, e_iota, _E), axis=1, keepdims=True)
    oh2 = e_iota == i2
    p2 = jnp.exp(m2 - m1)
    denom = 1.0 + p2
    gates = jnp.where(oh1, 1.0 / denom, 0.0) + jnp.where(oh2, p2 / denom, 0.0)

    s_row = jax.lax.broadcasted_iota(jnp.int32, (_E, _E * _R), 0)
    s_col = jax.lax.broadcasted_iota(jnp.int32, (_E, _E * _R), 1)
    sel = (s_col // _R == s_row).astype(jnp.float32)
    grep = jnp.dot(gates, sel, preferred_element_type=jnp.float32)

    h = jnp.dot(xb, aall_ref[...],
                preferred_element_type=jnp.float32)
    y = jnp.dot(h * grep, ball_ref[...],
                preferred_element_type=jnp.float32) * _SCALE
    tw = tw_ref[...]
    out_ref[...] = jnp.where(tw != 0.0, tw * y, 0.0)


def kernel(x, type_weight, Wg, A, Bw):
    n = _B * _S
    xf = x.reshape(n, _D)
    twf = type_weight.reshape(n, 1)
    a_all = jnp.transpose(A, (1, 0, 2)).reshape(_D, _E * _R)
    b_all = Bw.reshape(_E * _R, _D)
    y = pl.pallas_call(
        _moe_block,
        grid=(n // _BLK,),
        in_specs=[
            pl.BlockSpec((_BLK, _D), lambda i: (i, 0)),
            pl.BlockSpec((_BLK, 1), lambda i: (i, 0)),
            pl.BlockSpec((_D, _E), lambda i: (0, 0)),
            pl.BlockSpec((_D, _E * _R), lambda i: (0, 0)),
            pl.BlockSpec((_E * _R, _D), lambda i: (0, 0)),
        ],
        out_specs=pl.BlockSpec((_BLK, _D), lambda i: (i, 0)),
        out_shape=jax.ShapeDtypeStruct((n, _D), jnp.float32),
    )(xf, twf, Wg, a_all, b_all)
    return y.reshape(_B, _S, _D)


# speedup vs baseline: 8.0450x; 1.1148x over previous
import jax
import jax.numpy as jnp
from jax.experimental import pallas as pl

_B, _S, _D = 4, 4096, 2048
_E, _K, _R = 8, 2, 8
_SCALE = 16.0 / 8.0
_BLK = 256


def _moe_block(x_ref, tw_ref, wg_ref, aall_ref, ball_ref, out_ref):
    xb = x_ref[...]
    logits = jnp.dot(xb, wg_ref[...],
                     preferred_element_type=jnp.float32)

    e_iota = jax.lax.broadcasted_iota(jnp.int32, (_BLK, _E), 1)
    m1 = jnp.max(logits, axis=1, keepdims=True)
    i1 = jnp.min(jnp.where(logits == m1, e_iota, _E), axis=1, keepdims=True)
    masked = jnp.where(e_iota == i1, -jnp.inf, logits)
    m2 = jnp.max(masked, axis=1, keepdims=True)
    i2 = jnp.min(jnp.where(masked == m2, e_iota, _E), axis=1, keepdims=True)
    p2 = jnp.exp(m2 - m1)
    denom = 1.0 + p2

    l_exp = jax.lax.broadcasted_iota(jnp.int32, (_BLK, _E * _R), 1) // _R
    grep = (jnp.where(l_exp == i1, 1.0 / denom, 0.0)
            + jnp.where(l_exp == i2, p2 / denom, 0.0))

    h = jnp.dot(xb.astype(jnp.bfloat16), aall_ref[...],
                preferred_element_type=jnp.float32)
    y = jnp.dot((h * grep).astype(jnp.bfloat16), ball_ref[...],
                preferred_element_type=jnp.float32) * _SCALE
    out_ref[...] = tw_ref[...] * y


def kernel(x, type_weight, Wg, A, Bw):
    n = _B * _S
    xf = x.reshape(n, _D)
    twf = type_weight.reshape(n, 1)
    a_all = jnp.transpose(A, (1, 0, 2)).reshape(_D, _E * _R).astype(jnp.bfloat16)
    b_all = Bw.reshape(_E * _R, _D).astype(jnp.bfloat16)
    y = pl.pallas_call(
        _moe_block,
        grid=(n // _BLK,),
        in_specs=[
            pl.BlockSpec((_BLK, _D), lambda i: (i, 0)),
            pl.BlockSpec((_BLK, 1), lambda i: (i, 0)),
            pl.BlockSpec((_D, _E), lambda i: (0, 0)),
            pl.BlockSpec((_D, _E * _R), lambda i: (0, 0)),
            pl.BlockSpec((_E * _R, _D), lambda i: (0, 0)),
        ],
        out_specs=pl.BlockSpec((_BLK, _D), lambda i: (i, 0)),
        out_shape=jax.ShapeDtypeStruct((n, _D), jnp.float32),
    )(xf, twf, Wg, a_all, b_all)
    return y.reshape(_B, _S, _D)
